# baseline (speedup 1.0000x reference)
import functools

import jax
import jax.numpy as jnp
import numpy as np
from jax.experimental import pallas as pl
from jax.experimental.pallas import tpu as pltpu

QDIM = 4096
KDIM = 100000
DDIM = 128
TOPK = 16

BQ = 256
BK = 2048
NQB = QDIM // BQ
KPAD = ((KDIM + BK - 1) // BK) * BK
NKB = KPAD // BK

NEG = -3e38
IBIG = 2**31 - 1


def _cmp_swap(v, i, d):
    n = v.shape[-1]
    vr = v.reshape(v.shape[0], n // (2 * d), 2, d)
    ir = i.reshape(i.shape[0], n // (2 * d), 2, d)
    va, vb = vr[:, :, 0, :], vr[:, :, 1, :]
    ia, ib = ir[:, :, 0, :], ir[:, :, 1, :]
    take_a = (va > vb) | ((va == vb) & (ia < ib))
    hi_v = jnp.where(take_a, va, vb)
    hi_i = jnp.where(take_a, ia, ib)
    lo_v = jnp.where(take_a, vb, va)
    lo_i = jnp.where(take_a, ib, ia)
    v = jnp.stack([hi_v, lo_v], axis=2).reshape(v.shape)
    i = jnp.stack([hi_i, lo_i], axis=2).reshape(i.shape)
    return v, i


def _retrieve_kernel(q_ref, w_ref, b_ref, keys_ref, iota_ref, bias_ref,
                     vals_ref, idx_ref,
                     qn_ref, rv_ref, ri_ref):
    j = pl.program_id(0)
    i = pl.program_id(1)
    qrow = i * BQ

    @pl.when(j == 0)
    def _init():
        q = jax.lax.dot_general(q_ref[...], w_ref[...],
                                (((1,), (1,)), ((), ())),
                                preferred_element_type=jnp.float32)
        q = q + b_ref[...]
        nrm = jnp.sqrt(jnp.sum(q * q, axis=1, keepdims=True)) + 1e-8
        qn_ref[pl.ds(qrow, BQ), :] = q / nrm
        rv_ref[pl.ds(qrow, BQ), :] = jnp.full((BQ, TOPK), NEG, jnp.float32)
        ri_ref[pl.ds(qrow, BQ), :] = jnp.zeros((BQ, TOPK), jnp.int32)

    kb = keys_ref[...]
    knrm = jnp.sqrt(jnp.sum(kb * kb, axis=1, keepdims=True)) + 1e-8
    kn = kb / knrm
    qn = qn_ref[pl.ds(qrow, BQ), :]
    raw = jax.lax.dot_general(qn, kn, (((1,), (1,)), ((), ())),
                              preferred_element_type=jnp.float32)
    s = raw + bias_ref[...]

    gcol = iota_ref[...]

    bv, bi = [], []
    for _ in range(TOPK):
        m = jnp.max(s, axis=1, keepdims=True)
        gi = jnp.min(jnp.where(s == m, gcol, IBIG), axis=1, keepdims=True)
        s = jnp.where(gcol == gi, NEG, s)
        bv.append(m)
        bi.append(gi)
    bvv = jnp.concatenate(bv[::-1], axis=1)
    bii = jnp.concatenate(bi[::-1], axis=1)

    av = rv_ref[pl.ds(qrow, BQ), :]
    ai = ri_ref[pl.ds(qrow, BQ), :]
    take_a = (av > bvv) | ((av == bvv) & (ai < bii))
    newv = jnp.where(take_a, av, bvv)
    newi = jnp.where(take_a, ai, bii)
    for d in (8, 4, 2, 1):
        newv, newi = _cmp_swap(newv, newi, d)
    rv_ref[pl.ds(qrow, BQ), :] = newv
    ri_ref[pl.ds(qrow, BQ), :] = newi

    @pl.when(j == NKB - 1)
    def _emit():
        vals_ref[pl.ds(qrow, BQ), :] = newv
        idx_ref[pl.ds(qrow, BQ), :] = newi


@jax.jit
def _retrieve(queries, keys, W_q, b_q):
    keys_p = jnp.pad(keys, ((0, KPAD - KDIM), (0, 0)))
    b2 = b_q.reshape(1, DDIM)
    iota = jnp.arange(KPAD, dtype=jnp.int32).reshape(1, KPAD)
    bias = jnp.where(iota < KDIM, 0.0, NEG).astype(jnp.float32)
    grid = (NKB, NQB)
    out = pl.pallas_call(
        _retrieve_kernel,
        grid=grid,
        in_specs=[
            pl.BlockSpec((BQ, DDIM), lambda j, i: (i, 0)),
            pl.BlockSpec((DDIM, DDIM), lambda j, i: (0, 0)),
            pl.BlockSpec((1, DDIM), lambda j, i: (0, 0)),
            pl.BlockSpec((BK, DDIM), lambda j, i: (j, 0)),
            pl.BlockSpec((1, BK), lambda j, i: (0, j)),
            pl.BlockSpec((1, BK), lambda j, i: (0, j)),
        ],
        out_specs=[
            pl.BlockSpec((QDIM, TOPK), lambda j, i: (0, 0)),
            pl.BlockSpec((QDIM, TOPK), lambda j, i: (0, 0)),
        ],
        out_shape=[
            jax.ShapeDtypeStruct((QDIM, TOPK), jnp.float32),
            jax.ShapeDtypeStruct((QDIM, TOPK), jnp.int32),
        ],
        scratch_shapes=[
            pltpu.VMEM((QDIM, DDIM), jnp.float32),
            pltpu.VMEM((QDIM, TOPK), jnp.float32),
            pltpu.VMEM((QDIM, TOPK), jnp.int32),
        ],
        compiler_params=pltpu.CompilerParams(
            dimension_semantics=("arbitrary", "arbitrary")),
    )(queries, W_q, b2, keys_p, iota, bias)
    return out[0], out[1]


def kernel(queries, keys, W_q, b_q, k):
    vals, idx = _retrieve(queries, keys, W_q, b_q)
    k_arr = jnp.asarray(k)
    k_zero = k_arr - k_arr
    return (vals + k_zero.astype(vals.dtype),
            idx + k_zero.astype(idx.dtype))


# speedup vs baseline: 2.3824x; 2.3824x over previous
import functools

import jax
import jax.numpy as jnp
import numpy as np
from jax.experimental import pallas as pl
from jax.experimental.pallas import tpu as pltpu

QDIM = 4096
KDIM = 100000
DDIM = 128
TOPK = 16

BQ = 256
BK = 2048
NQB = QDIM // BQ
KPAD = ((KDIM + BK - 1) // BK) * BK
NKB = KPAD // BK

NEG = -3e38
IBIG = 2**31 - 1


def _retrieve_kernel(q_ref, w_ref, b_ref, keys_ref, iota_ref, bias_ref,
                     vals_ref, idx_ref,
                     qn_ref, rv_ref, ri_ref, s_ref, bv_ref, bi_ref):
    j = pl.program_id(0)
    i = pl.program_id(1)
    qrow = i * BQ

    @pl.when(j == 0)
    def _init():
        q = jax.lax.dot_general(q_ref[...], w_ref[...],
                                (((1,), (1,)), ((), ())),
                                preferred_element_type=jnp.float32)
        q = q + b_ref[...]
        nrm = jnp.sqrt(jnp.sum(q * q, axis=1, keepdims=True)) + 1e-8
        qn_ref[pl.ds(qrow, BQ), :] = q / nrm
        rv_ref[pl.ds(qrow, BQ), :] = jnp.full((BQ, TOPK), NEG, jnp.float32)
        ri_ref[pl.ds(qrow, BQ), :] = jnp.zeros((BQ, TOPK), jnp.int32)

    kb = keys_ref[...]
    knrm = jnp.sqrt(jnp.sum(kb * kb, axis=1, keepdims=True)) + 1e-8
    kn = kb / knrm
    qn = qn_ref[pl.ds(qrow, BQ), :]
    raw = jax.lax.dot_general(qn, kn, (((1,), (1,)), ((), ())),
                              preferred_element_type=jnp.float32)
    s = raw + bias_ref[...]
    s_ref[...] = s

    gcol = iota_ref[...]

    thr = rv_ref[pl.ds(qrow, BQ), :][:, TOPK - 1:TOPK]
    cnt = jnp.sum((s > thr).astype(jnp.int32), axis=1, keepdims=True)
    niter = jnp.minimum(jnp.max(cnt), TOPK)

    bv_ref[...] = jnp.full((BQ, TOPK), NEG, jnp.float32)
    bi_ref[...] = jnp.zeros((BQ, TOPK), jnp.int32)
    GRP = 4
    for g in range(TOPK // GRP):
        @pl.when(g * GRP < niter)
        def _group(g=g):
            sc = s_ref[...]
            for u in range(GRP):
                t = g * GRP + u
                m = jnp.max(sc, axis=1, keepdims=True)
                gi = jnp.min(jnp.where(sc == m, gcol, IBIG), axis=1,
                             keepdims=True)
                sc = jnp.where(gcol == gi, NEG, sc)
                bv_ref[:, t:t + 1] = m
                bi_ref[:, t:t + 1] = gi
            s_ref[...] = sc

    allv = jnp.concatenate([rv_ref[pl.ds(qrow, BQ), :], bv_ref[...]],
                           axis=1)
    alli = jnp.concatenate([ri_ref[pl.ds(qrow, BQ), :], bi_ref[...]],
                           axis=1)
    nv, ni = [], []
    for _ in range(TOPK):
        m = jnp.max(allv, axis=1, keepdims=True)
        hit = allv == m
        gi = jnp.min(jnp.where(hit, alli, IBIG), axis=1, keepdims=True)
        allv = jnp.where(hit & (alli == gi), NEG, allv)
        nv.append(m)
        ni.append(gi)
    newv = jnp.concatenate(nv, axis=1)
    newi = jnp.concatenate(ni, axis=1)
    rv_ref[pl.ds(qrow, BQ), :] = newv
    ri_ref[pl.ds(qrow, BQ), :] = newi

    @pl.when(j == NKB - 1)
    def _emit():
        vals_ref[pl.ds(qrow, BQ), :] = newv
        idx_ref[pl.ds(qrow, BQ), :] = newi


@jax.jit
def _retrieve(queries, keys, W_q, b_q):
    keys_p = jnp.pad(keys, ((0, KPAD - KDIM), (0, 0)))
    b2 = b_q.reshape(1, DDIM)
    iota = jnp.arange(KPAD, dtype=jnp.int32).reshape(1, KPAD)
    bias = jnp.where(iota < KDIM, 0.0, NEG).astype(jnp.float32)
    grid = (NKB, NQB)
    out = pl.pallas_call(
        _retrieve_kernel,
        grid=grid,
        in_specs=[
            pl.BlockSpec((BQ, DDIM), lambda j, i: (i, 0)),
            pl.BlockSpec((DDIM, DDIM), lambda j, i: (0, 0)),
            pl.BlockSpec((1, DDIM), lambda j, i: (0, 0)),
            pl.BlockSpec((BK, DDIM), lambda j, i: (j, 0)),
            pl.BlockSpec((1, BK), lambda j, i: (0, j)),
            pl.BlockSpec((1, BK), lambda j, i: (0, j)),
        ],
        out_specs=[
            pl.BlockSpec((QDIM, TOPK), lambda j, i: (0, 0)),
            pl.BlockSpec((QDIM, TOPK), lambda j, i: (0, 0)),
        ],
        out_shape=[
            jax.ShapeDtypeStruct((QDIM, TOPK), jnp.float32),
            jax.ShapeDtypeStruct((QDIM, TOPK), jnp.int32),
        ],
        scratch_shapes=[
            pltpu.VMEM((QDIM, DDIM), jnp.float32),
            pltpu.VMEM((QDIM, TOPK), jnp.float32),
            pltpu.VMEM((QDIM, TOPK), jnp.int32),
            pltpu.VMEM((BQ, BK), jnp.float32),
            pltpu.VMEM((BQ, TOPK), jnp.float32),
            pltpu.VMEM((BQ, TOPK), jnp.int32),
        ],
        compiler_params=pltpu.CompilerParams(
            dimension_semantics=("arbitrary", "arbitrary")),
    )(queries, W_q, b2, keys_p, iota, bias)
    return out[0], out[1]


def kernel(queries, keys, W_q, b_q, k):
    vals, idx = _retrieve(queries, keys, W_q, b_q)
    k_arr = jnp.asarray(k)
    k_zero = k_arr - k_arr
    return (vals + k_zero.astype(vals.dtype),
            idx + k_zero.astype(idx.dtype))


# speedup vs baseline: 3.1239x; 1.3112x over previous
import functools

import jax
import jax.numpy as jnp
import numpy as np
from jax.experimental import pallas as pl
from jax.experimental.pallas import tpu as pltpu

QDIM = 4096
KDIM = 100000
DDIM = 128
TOPK = 16

BQ = 256
BK = 2048
NQB = QDIM // BQ
KPAD = ((KDIM + BK - 1) // BK) * BK
NKB = KPAD // BK

NEG = -3e38
IBIG = 2**31 - 1


def _retrieve_kernel(q_ref, w_ref, b_ref, keys_ref, iota_ref, bias_ref,
                     vals_ref, idx_ref,
                     qn_ref, rv_ref, ri_ref, s_ref, bv_ref, bi_ref):
    j = pl.program_id(0)
    i = pl.program_id(1)
    qrow = i * BQ

    @pl.when(j == 0)
    def _init():
        q = jax.lax.dot_general(q_ref[...], w_ref[...],
                                (((1,), (1,)), ((), ())),
                                preferred_element_type=jnp.float32)
        q = q + b_ref[...]
        nrm = jnp.sqrt(jnp.sum(q * q, axis=1, keepdims=True)) + 1e-8
        qn_ref[pl.ds(qrow, BQ), :] = q / nrm
        rv_ref[pl.ds(qrow, BQ), :] = jnp.full((BQ, TOPK), NEG, jnp.float32)
        ri_ref[pl.ds(qrow, BQ), :] = jnp.zeros((BQ, TOPK), jnp.int32)

    kb = keys_ref[...]
    knrm = jnp.sqrt(jnp.sum(kb * kb, axis=1, keepdims=True)) + 1e-8
    kn = kb / knrm
    qn = qn_ref[pl.ds(qrow, BQ), :]
    raw = jax.lax.dot_general(qn, kn, (((1,), (1,)), ((), ())),
                              preferred_element_type=jnp.float32)
    s = raw + bias_ref[...]
    s_ref[...] = s

    gcol = iota_ref[...]

    thr = rv_ref[pl.ds(qrow, BQ), :][:, TOPK - 1:TOPK]
    cnt = jnp.sum((s > thr).astype(jnp.int32), axis=1, keepdims=True)
    niter = jnp.minimum(jnp.max(cnt), TOPK)

    bv_ref[...] = jnp.full((BQ, TOPK), NEG, jnp.float32)
    bi_ref[...] = jnp.zeros((BQ, TOPK), jnp.int32)
    GRP = 4
    for g in range(TOPK // GRP):
        @pl.when(g * GRP < niter)
        def _group(g=g):
            sc = s_ref[...]
            for u in range(GRP):
                t = g * GRP + u
                m = jnp.max(sc, axis=1, keepdims=True)
                gi = jnp.min(jnp.where(sc == m, gcol, IBIG), axis=1,
                             keepdims=True)
                sc = jnp.where(gcol == gi, NEG, sc)
                bv_ref[:, TOPK - 1 - t:TOPK - t] = m
                bi_ref[:, TOPK - 1 - t:TOPK - t] = gi
            s_ref[...] = sc

    lane16 = jax.lax.broadcasted_iota(jnp.int32, (BQ, TOPK), 1)
    av = rv_ref[pl.ds(qrow, BQ), :]
    ai = ri_ref[pl.ds(qrow, BQ), :]
    bvv = bv_ref[...]
    bii = bi_ref[...]
    bet = (av > bvv) | ((av == bvv) & (ai < bii))
    newv = jnp.where(bet, av, bvv)
    newi = jnp.where(bet, ai, bii)
    for d in (8, 4, 2, 1):
        low = (lane16 & d) == 0
        pv = jnp.where(low, pltpu.roll(newv, TOPK - d, 1),
                       pltpu.roll(newv, d, 1))
        pi = jnp.where(low, pltpu.roll(newi, TOPK - d, 1),
                       pltpu.roll(newi, d, 1))
        bet = (newv > pv) | ((newv == pv) & (newi < pi))
        keep = low == bet
        newv = jnp.where(keep, newv, pv)
        newi = jnp.where(keep, newi, pi)
    rv_ref[pl.ds(qrow, BQ), :] = newv
    ri_ref[pl.ds(qrow, BQ), :] = newi

    @pl.when(j == NKB - 1)
    def _emit():
        vals_ref[pl.ds(qrow, BQ), :] = newv
        idx_ref[pl.ds(qrow, BQ), :] = newi


@jax.jit
def _retrieve(queries, keys, W_q, b_q):
    keys_p = jnp.pad(keys, ((0, KPAD - KDIM), (0, 0)))
    b2 = b_q.reshape(1, DDIM)
    iota = jnp.arange(KPAD, dtype=jnp.int32).reshape(1, KPAD)
    bias = jnp.where(iota < KDIM, 0.0, NEG).astype(jnp.float32)
    grid = (NKB, NQB)
    out = pl.pallas_call(
        _retrieve_kernel,
        grid=grid,
        in_specs=[
            pl.BlockSpec((BQ, DDIM), lambda j, i: (i, 0)),
            pl.BlockSpec((DDIM, DDIM), lambda j, i: (0, 0)),
            pl.BlockSpec((1, DDIM), lambda j, i: (0, 0)),
            pl.BlockSpec((BK, DDIM), lambda j, i: (j, 0)),
            pl.BlockSpec((1, BK), lambda j, i: (0, j)),
            pl.BlockSpec((1, BK), lambda j, i: (0, j)),
        ],
        out_specs=[
            pl.BlockSpec((QDIM, TOPK), lambda j, i: (0, 0)),
            pl.BlockSpec((QDIM, TOPK), lambda j, i: (0, 0)),
        ],
        out_shape=[
            jax.ShapeDtypeStruct((QDIM, TOPK), jnp.float32),
            jax.ShapeDtypeStruct((QDIM, TOPK), jnp.int32),
        ],
        scratch_shapes=[
            pltpu.VMEM((QDIM, DDIM), jnp.float32),
            pltpu.VMEM((QDIM, TOPK), jnp.float32),
            pltpu.VMEM((QDIM, TOPK), jnp.int32),
            pltpu.VMEM((BQ, BK), jnp.float32),
            pltpu.VMEM((BQ, TOPK), jnp.float32),
            pltpu.VMEM((BQ, TOPK), jnp.int32),
        ],
        compiler_params=pltpu.CompilerParams(
            dimension_semantics=("arbitrary", "arbitrary")),
    )(queries, W_q, b2, keys_p, iota, bias)
    return out[0], out[1]


def kernel(queries, keys, W_q, b_q, k):
    vals, idx = _retrieve(queries, keys, W_q, b_q)
    k_arr = jnp.asarray(k)
    k_zero = k_arr - k_arr
    return (vals + k_zero.astype(vals.dtype),
            idx + k_zero.astype(idx.dtype))


# speedup vs baseline: 3.3187x; 1.0624x over previous
import functools

import jax
import jax.numpy as jnp
import numpy as np
from jax.experimental import pallas as pl
from jax.experimental.pallas import tpu as pltpu

QDIM = 4096
KDIM = 100000
DDIM = 128
TOPK = 16

BQ = 256
BK = 4096
NQB = QDIM // BQ
KPAD = ((KDIM + BK - 1) // BK) * BK
NKB = KPAD // BK

NEG = -3e38
IBIG = 2**31 - 1


def _retrieve_kernel(q_ref, w_ref, b_ref, keys_ref, iota_ref, bias_ref,
                     vals_ref, idx_ref,
                     qn_ref, rv_ref, ri_ref, s_ref, bv_ref, bi_ref,
                     kn_ref):
    j = pl.program_id(0)
    i = pl.program_id(1)
    qrow = i * BQ

    @pl.when(j == 0)
    def _init():
        q = jax.lax.dot_general(q_ref[...], w_ref[...],
                                (((1,), (1,)), ((), ())),
                                preferred_element_type=jnp.float32)
        q = q + b_ref[...]
        nrm = jnp.sqrt(jnp.sum(q * q, axis=1, keepdims=True)) + 1e-8
        qn_ref[pl.ds(qrow, BQ), :] = q / nrm
        rv_ref[pl.ds(qrow, BQ), :] = jnp.full((BQ, TOPK), NEG, jnp.float32)
        ri_ref[pl.ds(qrow, BQ), :] = jnp.zeros((BQ, TOPK), jnp.int32)

    @pl.when(i == 0)
    def _knorm():
        kb = keys_ref[...]
        knrm = jnp.sqrt(jnp.sum(kb * kb, axis=1, keepdims=True)) + 1e-8
        kn_ref[...] = kb / knrm

    qn = qn_ref[pl.ds(qrow, BQ), :]
    raw = jax.lax.dot_general(qn, kn_ref[...], (((1,), (1,)), ((), ())),
                              preferred_element_type=jnp.float32)
    s = raw + bias_ref[...]
    s_ref[...] = s

    gcol = iota_ref[...]

    thr = rv_ref[pl.ds(qrow, BQ), :][:, TOPK - 1:TOPK]
    cnt = jnp.sum((s > thr).astype(jnp.int32), axis=1, keepdims=True)
    niter = jnp.minimum(jnp.max(cnt), TOPK)

    bv_ref[...] = jnp.full((BQ, TOPK), NEG, jnp.float32)
    bi_ref[...] = jnp.zeros((BQ, TOPK), jnp.int32)
    GRP = 4
    for g in range(TOPK // GRP):
        @pl.when(g * GRP < niter)
        def _group(g=g):
            sc = s_ref[...]
            for u in range(GRP):
                t = g * GRP + u
                m = jnp.max(sc, axis=1, keepdims=True)
                gi = jnp.min(jnp.where(sc == m, gcol, IBIG), axis=1,
                             keepdims=True)
                sc = jnp.where(gcol == gi, NEG, sc)
                bv_ref[:, TOPK - 1 - t:TOPK - t] = m
                bi_ref[:, TOPK - 1 - t:TOPK - t] = gi
            s_ref[...] = sc

    lane16 = jax.lax.broadcasted_iota(jnp.int32, (BQ, TOPK), 1)
    av = rv_ref[pl.ds(qrow, BQ), :]
    ai = ri_ref[pl.ds(qrow, BQ), :]
    bvv = bv_ref[...]
    bii = bi_ref[...]
    bet = (av > bvv) | ((av == bvv) & (ai < bii))
    newv = jnp.where(bet, av, bvv)
    newi = jnp.where(bet, ai, bii)
    for d in (8, 4, 2, 1):
        low = (lane16 & d) == 0
        pv = jnp.where(low, pltpu.roll(newv, TOPK - d, 1),
                       pltpu.roll(newv, d, 1))
        pi = jnp.where(low, pltpu.roll(newi, TOPK - d, 1),
                       pltpu.roll(newi, d, 1))
        bet = (newv > pv) | ((newv == pv) & (newi < pi))
        keep = low == bet
        newv = jnp.where(keep, newv, pv)
        newi = jnp.where(keep, newi, pi)
    rv_ref[pl.ds(qrow, BQ), :] = newv
    ri_ref[pl.ds(qrow, BQ), :] = newi

    @pl.when(j == NKB - 1)
    def _emit():
        vals_ref[pl.ds(qrow, BQ), :] = newv
        idx_ref[pl.ds(qrow, BQ), :] = newi


@jax.jit
def _retrieve(queries, keys, W_q, b_q):
    keys_p = jnp.pad(keys, ((0, KPAD - KDIM), (0, 0)))
    b2 = b_q.reshape(1, DDIM)
    iota = jnp.arange(KPAD, dtype=jnp.int32).reshape(1, KPAD)
    bias = jnp.where(iota < KDIM, 0.0, NEG).astype(jnp.float32)
    grid = (NKB, NQB)
    out = pl.pallas_call(
        _retrieve_kernel,
        grid=grid,
        in_specs=[
            pl.BlockSpec((BQ, DDIM), lambda j, i: (i, 0)),
            pl.BlockSpec((DDIM, DDIM), lambda j, i: (0, 0)),
            pl.BlockSpec((1, DDIM), lambda j, i: (0, 0)),
            pl.BlockSpec((BK, DDIM), lambda j, i: (j, 0)),
            pl.BlockSpec((1, BK), lambda j, i: (0, j)),
            pl.BlockSpec((1, BK), lambda j, i: (0, j)),
        ],
        out_specs=[
            pl.BlockSpec((QDIM, TOPK), lambda j, i: (0, 0)),
            pl.BlockSpec((QDIM, TOPK), lambda j, i: (0, 0)),
        ],
        out_shape=[
            jax.ShapeDtypeStruct((QDIM, TOPK), jnp.float32),
            jax.ShapeDtypeStruct((QDIM, TOPK), jnp.int32),
        ],
        scratch_shapes=[
            pltpu.VMEM((QDIM, DDIM), jnp.float32),
            pltpu.VMEM((QDIM, TOPK), jnp.float32),
            pltpu.VMEM((QDIM, TOPK), jnp.int32),
            pltpu.VMEM((BQ, BK), jnp.float32),
            pltpu.VMEM((BQ, TOPK), jnp.float32),
            pltpu.VMEM((BQ, TOPK), jnp.int32),
            pltpu.VMEM((BK, DDIM), jnp.float32),
        ],
        compiler_params=pltpu.CompilerParams(
            dimension_semantics=("arbitrary", "arbitrary")),
    )(queries, W_q, b2, keys_p, iota, bias)
    return out[0], out[1]


def kernel(queries, keys, W_q, b_q, k):
    vals, idx = _retrieve(queries, keys, W_q, b_q)
    k_arr = jnp.asarray(k)
    k_zero = k_arr - k_arr
    return (vals + k_zero.astype(vals.dtype),
            idx + k_zero.astype(idx.dtype))
